# baseline (speedup 1.0000x reference)
import functools

import jax
import jax.numpy as jnp
from jax import lax
from jax.experimental import pallas as pl
from jax.experimental.pallas import tpu as pltpu
from jax.experimental.pallas import tpu_sc as plsc

B, T, D = 16, 2048, 64
VOCAB = 1000000
HALF = VOCAB // 2
N = B * T
NC, NS = 2, 16
NW = NC * NS
PER_W = N // NW
CH = 128
NCH = PER_W // CH
VPR = D // 16

_mesh = plsc.VectorSubcoreMesh(core_axis_name="c", subcore_axis_name="s")


@functools.partial(
    pl.kernel,
    mesh=_mesh,
    compiler_params=pltpu.CompilerParams(use_tc_tiling_on_sc=False),
    out_type=jax.ShapeDtypeStruct((N, D), jnp.float32),
    scratch_types=[
        pltpu.VMEM((PER_W,), jnp.int32),
        pltpu.VMEM((PER_W,), jnp.int32),
        pltpu.VMEM((PER_W,), jnp.int32),
        pltpu.VMEM((CH, D), jnp.float32),
        pltpu.VMEM((CH, D), jnp.float32),
        pltpu.VMEM((CH, D), jnp.float32),
        pltpu.VMEM((CH, D), jnp.float32),
        pltpu.SemaphoreType.DMA,
        pltpu.SemaphoreType.DMA,
    ],
)
def _embed_sc(x_hbm, wt0_hbm, wt1_hbm, pt_hbm, out_hbm, idx_v, i0_v, i1_v,
              r0_v, r1_v, pos_v, out_v, sem0, sem1):
    wid = lax.axis_index("s") * NC + lax.axis_index("c")
    base = wid * PER_W
    pltpu.sync_copy(x_hbm.at[pl.ds(base, PER_W)], idx_v)

    def clamp(j, _):
        sl = pl.ds(j * 16, 16)
        v = idx_v[sl]
        i0_v[sl] = jnp.minimum(v, HALF - 1)
        i1_v[sl] = jnp.minimum(jnp.maximum(v - HALF, 0), HALF - 1)
        return ()

    lax.fori_loop(0, PER_W // 16, clamp, ())

    def chunk(i, _):
        pltpu.async_copy(
            wt0_hbm.at[i0_v.at[pl.ds(i * CH, CH)]], r0_v, sem0
        )
        pltpu.async_copy(
            wt1_hbm.at[i1_v.at[pl.ds(i * CH, CH)]], r1_v, sem1
        )
        pltpu.sync_copy(
            pt_hbm.at[pl.ds(lax.rem(base + i * CH, T), CH)], pos_v
        )
        pltpu.make_async_copy(
            wt0_hbm.at[i0_v.at[pl.ds(i * CH, CH)]], r0_v, sem0
        ).wait()
        pltpu.make_async_copy(
            wt1_hbm.at[i1_v.at[pl.ds(i * CH, CH)]], r1_v, sem1
        ).wait()

        def row(r, _):
            v16 = idx_v[pl.ds(i * CH + (r // 16) * 16, 16)]
            return ()

        for g in range(CH // 16):
            v16 = idx_v[pl.ds(i * CH + g * 16, 16)]
            for l in range(16):
                r = g * 16 + l
                w = jnp.where(v16[l] < HALF, 1.0, 0.0)
                wv = jnp.zeros((16,), jnp.float32) + w
                for c in range(VPR):
                    sl = pl.ds(c * 16, 16)
                    a = r0_v[r, sl]
                    bb = r1_v[r, sl]
                    out_v[r, sl] = (a - bb) * wv + bb + pos_v[r, sl]
        pltpu.sync_copy(out_v, out_hbm.at[pl.ds(base + i * CH, CH)])
        return ()

    lax.fori_loop(0, NCH, chunk, ())


def kernel(x, word_table, pos_table):
    flat = _embed_sc(
        x.reshape(N).astype(jnp.int32),
        word_table[:HALF],
        word_table[HALF:],
        pos_table,
    )
    return flat.reshape(B, T, D)
